# Initial kernel scaffold; baseline (speedup 1.0000x reference)
import jax
import jax.numpy as jnp
from jax.experimental import pallas as pl


def kernel(sigma, embed_table):
    raise NotImplementedError("write your pallas kernel here")


# speedup vs baseline: 4.5474x; 4.5474x over previous
import functools

import jax
import jax.numpy as jnp
from jax.experimental import pallas as pl
from jax.experimental.pallas import tpu as pltpu

_BB = 128


def _body(sig_ref, sigr_ref, tab_ref, out1_ref, out2_ref, states_ref):
    sig = sig_ref[...]
    s_f = (sig + 1.0) * 0.5
    s_r = (sigr_ref[...] + 1.0) * 0.5
    states_ref[...] = s_f.astype(jnp.int32)

    w0 = tab_ref[0, :][None, None, :]
    w1 = tab_ref[1, :][None, None, :]
    w2 = tab_ref[2, :][None, None, :]

    bb = sig.shape[0]
    two = jnp.full((bb, 1), 2.0, sig.dtype)
    s1 = jnp.concatenate([two, s_f], axis=1)
    s2 = jnp.concatenate([two, s_r], axis=1)

    def emb(s):
        se = s[:, :, None]
        return jnp.where(se == 0.0, w0, jnp.where(se == 1.0, w1, w2))

    out1_ref[...] = emb(s1)
    out2_ref[...] = emb(s2)


@jax.jit
def kernel(sigma, embed_table):
    batch, n = sigma.shape
    feat = embed_table.shape[1]
    grid = (batch // _BB,)
    out1, out2, states = pl.pallas_call(
        _body,
        grid=grid,
        in_specs=[
            pl.BlockSpec((_BB, n), lambda i: (i, 0)),
            pl.BlockSpec((_BB, n), lambda i: (i, 0)),
            pl.BlockSpec((3, feat), lambda i: (0, 0)),
        ],
        out_specs=[
            pl.BlockSpec((_BB, n + 1, feat), lambda i: (i, 0, 0)),
            pl.BlockSpec((_BB, n + 1, feat), lambda i: (i, 0, 0)),
            pl.BlockSpec((_BB, n), lambda i: (i, 0)),
        ],
        out_shape=[
            jax.ShapeDtypeStruct((batch, n + 1, feat), jnp.float32),
            jax.ShapeDtypeStruct((batch, n + 1, feat), jnp.float32),
            jax.ShapeDtypeStruct((batch, n), jnp.int32),
        ],
        compiler_params=pltpu.CompilerParams(
            dimension_semantics=("arbitrary",),
        ),
    )(sigma, sigma[:, ::-1], embed_table)
    return (out1, out2, states)
